# baseline (speedup 1.0000x reference)
import functools
import jax
import jax.numpy as jnp
from jax import lax
from jax.experimental import pallas as pl
from jax.experimental.pallas import tpu as pltpu
from jax.experimental.pallas import tpu_sc as plsc

_B, _P, _D = 64, 576, 768
_NC, _NS = 2, 16
_NW = _NC * _NS
_PW = _P // _NW
_CH = _PW * _D
_NV = _CH // 16
_GB = 2
_NG = _B // _GB

_mesh = plsc.VectorSubcoreMesh(core_axis_name="c", subcore_axis_name="s")


@functools.partial(
    pl.kernel,
    mesh=_mesh,
    out_type=jax.ShapeDtypeStruct((_NG, _GB, _P * _D), jnp.float32),
    scratch_types=[
        pltpu.VMEM((_CH,), jnp.float32),
        pltpu.VMEM((_GB, _CH), jnp.float32),
        pltpu.VMEM((_GB, _CH), jnp.float32),
        pltpu.VMEM((_GB, _CH), jnp.float32),
        pltpu.VMEM((_GB, _CH), jnp.float32),
        pltpu.SemaphoreType.DMA,
        pltpu.SemaphoreType.DMA,
        pltpu.SemaphoreType.DMA,
        pltpu.SemaphoreType.DMA,
    ],
)
def _sc_add(enc_hbm, pos_hbm, out_hbm, pos_v, ib0, ib1, ob0, ob1, si0, si1, so0, so1):
    wid = lax.axis_index("s") * _NC + lax.axis_index("c")
    base = wid * _CH
    sl = pl.ds(base, _CH)
    pltpu.sync_copy(pos_hbm.at[sl], pos_v)

    pltpu.async_copy(enc_hbm.at[0, :, sl], ib0, si0)
    pltpu.async_copy(enc_hbm.at[1, :, sl], ib1, si1)

    def halfstep(g, ib, ob, si, so):
        pltpu.make_async_copy(enc_hbm.at[g, :, sl], ib, si).wait()

        @pl.when(g >= 2)
        def _():
            pltpu.make_async_copy(ob, out_hbm.at[g, :, sl], so).wait()

        for j in range(_GB):
            @plsc.parallel_loop(0, _NV, step=1, unroll=8)
            def _(i):
                s = pl.ds(i * 16, 16)
                ob[j, s] = ib[j, s] + pos_v[s]

        pltpu.async_copy(ob, out_hbm.at[g, :, sl], so)

        @pl.when(g + 2 < _NG)
        def _():
            pltpu.async_copy(enc_hbm.at[g + 2, :, sl], ib, si)

    def body(i, carry):
        halfstep(2 * i, ib0, ob0, si0, so0)
        halfstep(2 * i + 1, ib1, ob1, si1, so1)
        return carry

    lax.fori_loop(0, _NG // 2, body, 0)

    pltpu.make_async_copy(ob0, out_hbm.at[_NG - 2, :, sl], so0).wait()
    pltpu.make_async_copy(ob1, out_hbm.at[_NG - 1, :, sl], so1).wait()


def kernel(encoded_patches, pos_table):
    enc3 = encoded_patches.reshape(_NG, _GB, _P * _D)
    pos1 = pos_table.reshape(_P * _D)
    out = _sc_add(enc3, pos1)
    return out.reshape(_B, _P, _D)


# speedup vs baseline: 1.9458x; 1.9458x over previous
import functools
import jax
import jax.numpy as jnp
from jax import lax
from jax.experimental import pallas as pl
from jax.experimental.pallas import tpu as pltpu
from jax.experimental.pallas import tpu_sc as plsc

_B, _P, _D = 64, 576, 768
_NC, _NS = 2, 16
_NW = _NC * _NS
_PW = _P // _NW
_CH = _PW * _D
_NV = _CH // 16
_NBUF = 8
_PD = 6

_mesh = plsc.VectorSubcoreMesh(core_axis_name="c", subcore_axis_name="s")


@functools.partial(
    pl.kernel,
    mesh=_mesh,
    out_type=jax.ShapeDtypeStruct((_B, _P * _D), jnp.float32),
    scratch_types=(
        [pltpu.VMEM((_CH,), jnp.float32)]
        + [pltpu.VMEM((_CH,), jnp.float32)] * _NBUF
        + [pltpu.SemaphoreType.DMA] * _NBUF
        + [pltpu.SemaphoreType.DMA] * _NBUF
    ),
)
def _sc_add(enc_hbm, pos_hbm, out_hbm, pos_v, *rest):
    bufs = rest[:_NBUF]
    sin = rest[_NBUF:2 * _NBUF]
    sout = rest[2 * _NBUF:3 * _NBUF]

    wid = lax.axis_index("s") * _NC + lax.axis_index("c")
    base = wid * _CH
    sl = pl.ds(base, _CH)
    pltpu.sync_copy(pos_hbm.at[sl], pos_v)

    for k in range(_PD):
        pltpu.async_copy(enc_hbm.at[k, sl], bufs[k], sin[k])

    def body(i, carry):
        h0 = i * _NBUF
        for k in range(_NBUF):
            h = h0 + k
            buf, si, so = bufs[k], sin[k], sout[k]
            pltpu.make_async_copy(enc_hbm.at[h, sl], buf, si).wait()

            @plsc.parallel_loop(0, _NV, step=1, unroll=8)
            def _(i2):
                s = pl.ds(i2 * 16, 16)
                buf[s] = buf[s] + pos_v[s]

            pltpu.async_copy(buf, out_hbm.at[h, sl], so)

            k2 = (k + _PD) % _NBUF
            @pl.when(h + _PD < _B)
            def _():
                @pl.when(h >= 2)
                def _():
                    pltpu.make_async_copy(
                        bufs[k2], out_hbm.at[h - 2, sl], sout[k2]
                    ).wait()

                pltpu.async_copy(enc_hbm.at[h + _PD, sl], bufs[k2], sin[k2])
        return carry

    lax.fori_loop(0, _B // _NBUF, body, 0)

    for t in range(_NBUF):
        h = _B - _NBUF + t
        k = h % _NBUF
        pltpu.make_async_copy(bufs[k], out_hbm.at[h, sl], sout[k]).wait()


def kernel(encoded_patches, pos_table):
    enc2 = encoded_patches.reshape(_B, _P * _D)
    pos1 = pos_table.reshape(_P * _D)
    out = _sc_add(enc2, pos1)
    return out.reshape(_B, _P, _D)
